# Initial kernel scaffold; baseline (speedup 1.0000x reference)
import jax
import jax.numpy as jnp
from jax.experimental import pallas as pl


def kernel(x_skill, x_job, x_resume, edge_index_skill_job, edge_index_skill_resume, W_rel_sj, b_rel_sj, W_root_sj, W_rel_sr, b_rel_sr, W_root_sr, Wm1, bm1, Wm2, bm2, Wm3, bm3):
    raise NotImplementedError("write your pallas kernel here")


# speedup vs baseline: 6.7905x; 6.7905x over previous
import functools

import jax
import jax.numpy as jnp
from jax import lax
from jax.experimental import pallas as pl
from jax.experimental.pallas import tpu as pltpu
from jax.experimental.pallas import tpu_sc as plsc

N = 10000
E = 320000
D = 128
H1 = 512
H2 = 256

CHUNK = 128
GRP = 8
NGRP = -(-E // (CHUNK * GRP))
E_PAD = NGRP * CHUNK * GRP
NTILES = 16
GPT = -(-NGRP // NTILES)
N_ACC = N + 8
WRT = 624
WTAIL = N - NTILES * WRT


def _sc_agg_body(x_hbm, src_hbm, dst_hbm, zeros_hbm, out_hbm,
                 sidx_v, didx_v, rows_v, acc_sh, sem):
    r = lax.axis_index("c")
    s = lax.axis_index("s")

    zstart = s * WRT
    pltpu.sync_copy(zeros_hbm.at[pl.ds(0, WRT)], acc_sh.at[pl.ds(zstart, WRT)])

    @pl.when(s == NTILES - 1)
    def _():
        pltpu.sync_copy(zeros_hbm.at[pl.ds(0, WTAIL)],
                        acc_sh.at[pl.ds(NTILES * WRT, WTAIL)])

    plsc.subcore_barrier()

    def _step(i, carry):
        g = i * NTILES + s

        @pl.when(g < NGRP)
        def _():
            pltpu.sync_copy(src_hbm.at[r, g], sidx_v)
            pltpu.sync_copy(dst_hbm.at[r, g], didx_v)
            for j in range(GRP):
                pltpu.async_copy(x_hbm.at[sidx_v.at[j]], rows_v, sem).wait()
                pltpu.sync_copy(rows_v, acc_sh.at[didx_v.at[j]], add=True)

        return carry

    lax.fori_loop(0, GPT, _step, 0)
    plsc.subcore_barrier()

    pltpu.sync_copy(acc_sh.at[pl.ds(zstart, WRT)],
                    out_hbm.at[r, pl.ds(zstart, WRT)])

    @pl.when(s == NTILES - 1)
    def _():
        pltpu.sync_copy(acc_sh.at[pl.ds(NTILES * WRT, WTAIL)],
                        out_hbm.at[r, pl.ds(NTILES * WRT, WTAIL)])


_sc_agg = functools.partial(
    pl.kernel,
    out_type=jax.ShapeDtypeStruct((2, N, D), jnp.float32),
    mesh=plsc.VectorSubcoreMesh(core_axis_name="c", subcore_axis_name="s"),
    scratch_types=[
        pltpu.VMEM((GRP, CHUNK), jnp.int32),
        pltpu.VMEM((GRP, CHUNK), jnp.int32),
        pltpu.VMEM((CHUNK, D), jnp.float32),
        pltpu.VMEM_SHARED((N_ACC, D), jnp.float32),
        pltpu.SemaphoreType.DMA,
    ],
)(_sc_agg_body)


RB = 1000


def _tc_body(aggj_ref, xj_ref, aggr_ref, xr_ref,
             wrelj_ref, wrootj_ref, bj_ref,
             wrelr_ref, wrootr_ref, br_ref,
             wm1a_ref, wm1b_ref, bm1_ref,
             wm2_ref, bm2_ref, wm3_ref, bm3_ref, out_ref):
    f32 = jnp.float32
    hj = (jnp.dot(aggj_ref[...], wrelj_ref[...], preferred_element_type=f32)
          + jnp.dot(xj_ref[...], wrootj_ref[...], preferred_element_type=f32)
          + bj_ref[...])
    hj = jnp.maximum(hj, 0.0)
    hr = (jnp.dot(aggr_ref[...], wrelr_ref[...], preferred_element_type=f32)
          + jnp.dot(xr_ref[...], wrootr_ref[...], preferred_element_type=f32)
          + br_ref[...])
    hr = jnp.maximum(hr, 0.0)
    h1 = (jnp.dot(hj, wm1a_ref[...], preferred_element_type=f32)
          + jnp.dot(hr, wm1b_ref[...], preferred_element_type=f32)
          + bm1_ref[...])
    h1 = jnp.maximum(h1, 0.0)
    h2 = jnp.maximum(
        jnp.dot(h1, wm2_ref[...], preferred_element_type=f32) + bm2_ref[...],
        0.0)
    out_ref[...] = (jnp.sum(h2 * wm3_ref[...], axis=1, keepdims=True)
                    + bm3_ref[...])


def _full_spec(shape):
    return pl.BlockSpec(shape, lambda i: (0,) * len(shape))


_tc_epilogue = pl.pallas_call(
    _tc_body,
    grid=(N // RB,),
    in_specs=[
        pl.BlockSpec((RB, D), lambda i: (i, 0)),
        pl.BlockSpec((RB, D), lambda i: (i, 0)),
        pl.BlockSpec((RB, D), lambda i: (i, 0)),
        pl.BlockSpec((RB, D), lambda i: (i, 0)),
        _full_spec((D, D)), _full_spec((D, D)), _full_spec((1, D)),
        _full_spec((D, D)), _full_spec((D, D)), _full_spec((1, D)),
        _full_spec((D, H1)), _full_spec((D, H1)), _full_spec((1, H1)),
        _full_spec((H1, H2)), _full_spec((1, H2)),
        _full_spec((1, H2)), _full_spec((1, 1)),
    ],
    out_specs=pl.BlockSpec((RB, 1), lambda i: (i, 0)),
    out_shape=jax.ShapeDtypeStruct((N, 1), jnp.float32),
)


def _pad_idx(idx, fill):
    pad = jnp.full((E_PAD - E,), fill, jnp.int32)
    return jnp.concatenate([idx, pad]).reshape(NGRP, GRP, CHUNK)


def kernel(x_skill, x_job, x_resume, edge_index_skill_job,
           edge_index_skill_resume, W_rel_sj, b_rel_sj, W_root_sj, W_rel_sr,
           b_rel_sr, W_root_sr, Wm1, bm1, Wm2, bm2, Wm3, bm3):
    src_all = jnp.stack([_pad_idx(edge_index_skill_job[0], 0),
                         _pad_idx(edge_index_skill_resume[0], 0)])
    dst_all = jnp.stack([_pad_idx(edge_index_skill_job[1], N),
                         _pad_idx(edge_index_skill_resume[1], N)])
    zeros = jnp.zeros((WRT, D), jnp.float32)
    agg = _sc_agg(x_skill, src_all, dst_all, zeros)

    out = _tc_epilogue(
        agg[0], x_job, agg[1], x_resume,
        W_rel_sj.T, W_root_sj.T, b_rel_sj.reshape(1, D),
        W_rel_sr.T, W_root_sr.T, b_rel_sr.reshape(1, D),
        Wm1.T[:D], Wm1.T[D:], bm1.reshape(1, H1),
        Wm2.T, bm2.reshape(1, H2),
        Wm3, bm3.reshape(1, 1),
    )
    return out.reshape(N)


# speedup vs baseline: 8.8548x; 1.3040x over previous
import functools

import jax
import jax.numpy as jnp
from jax import lax
from jax.experimental import pallas as pl
from jax.experimental.pallas import tpu as pltpu
from jax.experimental.pallas import tpu_sc as plsc

N = 10000
E = 320000
D = 128
H1 = 512
H2 = 256

CHUNK = 128
GRP = 8
NGRP = -(-E // (CHUNK * GRP))
E_PAD = NGRP * CHUNK * GRP
NTILES = 16
GPT = -(-NGRP // NTILES)
N_ACC = N + 8
WRT = 624
WTAIL = N - NTILES * WRT


def _sc_agg_body(x_hbm, src_hbm, dst_hbm, zeros_hbm, out_hbm,
                 sidx2, didx2, rows2, acc_sh, isem, gsem, ssem):
    r = lax.axis_index("c")
    s = lax.axis_index("s")
    n_my = (NGRP - 1 - s) // NTILES + 1
    t_total = n_my * GRP

    zstart = s * WRT
    pltpu.sync_copy(zeros_hbm.at[pl.ds(0, WRT)], acc_sh.at[pl.ds(zstart, WRT)])

    @pl.when(s == NTILES - 1)
    def _():
        pltpu.sync_copy(zeros_hbm.at[pl.ds(0, WTAIL)],
                        acc_sh.at[pl.ds(NTILES * WRT, WTAIL)])

    pltpu.sync_copy(src_hbm.at[r, s], sidx2.at[0])
    pltpu.sync_copy(dst_hbm.at[r, s], didx2.at[0])
    plsc.subcore_barrier()

    pltpu.async_copy(x_hbm.at[sidx2.at[0, 0]], rows2.at[0], gsem)

    def _step(t, carry):
        i, j = lax.div(t, GRP), lax.rem(t, GRP)
        b = lax.rem(t, 2)
        pltpu.make_async_copy(x_hbm.at[pl.ds(0, CHUNK)],
                              rows2.at[b], gsem).wait()
        pltpu.async_copy(rows2.at[b], acc_sh.at[didx2.at[lax.rem(i, 2), j]],
                         ssem, add=True)

        @pl.when(t >= 1)
        def _():
            pltpu.make_async_copy(x_hbm.at[pl.ds(0, CHUNK)],
                                  rows2.at[0], ssem).wait()

        @pl.when(jnp.logical_and(j == 0, i + 1 < n_my))
        def _():
            g = (i + 1) * NTILES + s
            ip = lax.rem(i + 1, 2)
            pltpu.async_copy(src_hbm.at[r, g], sidx2.at[ip], isem)
            pltpu.async_copy(dst_hbm.at[r, g], didx2.at[ip], isem)

        @pl.when(t + 1 < t_total)
        def _():
            tn = t + 1
            i2, j2 = lax.div(tn, GRP), lax.rem(tn, GRP)

            @pl.when(j2 == 0)
            def _():
                pltpu.make_async_copy(src_hbm.at[r, s], sidx2.at[0],
                                      isem).wait()
                pltpu.make_async_copy(dst_hbm.at[r, s], didx2.at[0],
                                      isem).wait()

            pltpu.async_copy(x_hbm.at[sidx2.at[lax.rem(i2, 2), j2]],
                             rows2.at[lax.rem(tn, 2)], gsem)

        return carry

    lax.fori_loop(0, t_total, _step, 0)
    pltpu.make_async_copy(x_hbm.at[pl.ds(0, CHUNK)], rows2.at[0], ssem).wait()
    plsc.subcore_barrier()

    pltpu.sync_copy(acc_sh.at[pl.ds(zstart, WRT)],
                    out_hbm.at[r, pl.ds(zstart, WRT)])

    @pl.when(s == NTILES - 1)
    def _():
        pltpu.sync_copy(acc_sh.at[pl.ds(NTILES * WRT, WTAIL)],
                        out_hbm.at[r, pl.ds(NTILES * WRT, WTAIL)])


_sc_agg = functools.partial(
    pl.kernel,
    out_type=jax.ShapeDtypeStruct((2, N, D), jnp.float32),
    mesh=plsc.VectorSubcoreMesh(core_axis_name="c", subcore_axis_name="s"),
    scratch_types=[
        pltpu.VMEM((2, GRP, CHUNK), jnp.int32),
        pltpu.VMEM((2, GRP, CHUNK), jnp.int32),
        pltpu.VMEM((2, CHUNK, D), jnp.float32),
        pltpu.VMEM_SHARED((N_ACC, D), jnp.float32),
        pltpu.SemaphoreType.DMA,
        pltpu.SemaphoreType.DMA,
        pltpu.SemaphoreType.DMA,
    ],
)(_sc_agg_body)


RB = 1000


def _tc_body(aggj_ref, xj_ref, aggr_ref, xr_ref,
             wrelj_ref, wrootj_ref, bj_ref,
             wrelr_ref, wrootr_ref, br_ref,
             wm1a_ref, wm1b_ref, bm1_ref,
             wm2_ref, bm2_ref, wm3_ref, bm3_ref, out_ref):
    f32 = jnp.float32
    hj = (jnp.dot(aggj_ref[...], wrelj_ref[...], preferred_element_type=f32)
          + jnp.dot(xj_ref[...], wrootj_ref[...], preferred_element_type=f32)
          + bj_ref[...])
    hj = jnp.maximum(hj, 0.0)
    hr = (jnp.dot(aggr_ref[...], wrelr_ref[...], preferred_element_type=f32)
          + jnp.dot(xr_ref[...], wrootr_ref[...], preferred_element_type=f32)
          + br_ref[...])
    hr = jnp.maximum(hr, 0.0)
    h1 = (jnp.dot(hj, wm1a_ref[...], preferred_element_type=f32)
          + jnp.dot(hr, wm1b_ref[...], preferred_element_type=f32)
          + bm1_ref[...])
    h1 = jnp.maximum(h1, 0.0)
    h2 = jnp.maximum(
        jnp.dot(h1, wm2_ref[...], preferred_element_type=f32) + bm2_ref[...],
        0.0)
    out_ref[...] = (jnp.sum(h2 * wm3_ref[...], axis=1, keepdims=True)
                    + bm3_ref[...])


def _full_spec(shape):
    return pl.BlockSpec(shape, lambda i: (0,) * len(shape))


_tc_epilogue = pl.pallas_call(
    _tc_body,
    grid=(N // RB,),
    in_specs=[
        pl.BlockSpec((RB, D), lambda i: (i, 0)),
        pl.BlockSpec((RB, D), lambda i: (i, 0)),
        pl.BlockSpec((RB, D), lambda i: (i, 0)),
        pl.BlockSpec((RB, D), lambda i: (i, 0)),
        _full_spec((D, D)), _full_spec((D, D)), _full_spec((1, D)),
        _full_spec((D, D)), _full_spec((D, D)), _full_spec((1, D)),
        _full_spec((D, H1)), _full_spec((D, H1)), _full_spec((1, H1)),
        _full_spec((H1, H2)), _full_spec((1, H2)),
        _full_spec((1, H2)), _full_spec((1, 1)),
    ],
    out_specs=pl.BlockSpec((RB, 1), lambda i: (i, 0)),
    out_shape=jax.ShapeDtypeStruct((N, 1), jnp.float32),
)


def _pad_idx(idx, fill):
    pad = jnp.full((E_PAD - E,), fill, jnp.int32)
    return jnp.concatenate([idx, pad]).reshape(NGRP, GRP, CHUNK)


def kernel(x_skill, x_job, x_resume, edge_index_skill_job,
           edge_index_skill_resume, W_rel_sj, b_rel_sj, W_root_sj, W_rel_sr,
           b_rel_sr, W_root_sr, Wm1, bm1, Wm2, bm2, Wm3, bm3):
    src_all = jnp.stack([_pad_idx(edge_index_skill_job[0], 0),
                         _pad_idx(edge_index_skill_resume[0], 0)])
    dst_all = jnp.stack([_pad_idx(edge_index_skill_job[1], N),
                         _pad_idx(edge_index_skill_resume[1], N)])
    zeros = jnp.zeros((WRT, D), jnp.float32)
    agg = _sc_agg(x_skill, src_all, dst_all, zeros)

    out = _tc_epilogue(
        agg[0], x_job, agg[1], x_resume,
        W_rel_sj.T, W_root_sj.T, b_rel_sj.reshape(1, D),
        W_rel_sr.T, W_root_sr.T, b_rel_sr.reshape(1, D),
        Wm1.T[:D], Wm1.T[D:], bm1.reshape(1, H1),
        Wm2.T, bm2.reshape(1, H2),
        Wm3, bm3.reshape(1, 1),
    )
    return out.reshape(N)


# speedup vs baseline: 9.6436x; 1.0891x over previous
import functools

import jax
import jax.numpy as jnp
from jax import lax
from jax.experimental import pallas as pl
from jax.experimental.pallas import tpu as pltpu
from jax.experimental.pallas import tpu_sc as plsc

N = 10000
E = 320000
D = 128
H1 = 512
H2 = 256

CHUNK = 112
GRP = 8
NGRP = -(-E // (CHUNK * GRP))
E_PAD = NGRP * CHUNK * GRP
NTILES = 16
GPT = -(-NGRP // NTILES)
N_ACC = N + 8
WRT = 624
WTAIL = N - NTILES * WRT


def _sc_agg_body(x_hbm, src_hbm, dst_hbm, zeros_hbm, out_hbm,
                 sidx2, didx2, rows2, acc_sh, isem, gsem, ssem):
    r = lax.axis_index("c")
    s = lax.axis_index("s")
    n_my = (NGRP - 1 - s) // NTILES + 1
    t_total = n_my * GRP

    zstart = s * WRT
    pltpu.sync_copy(zeros_hbm.at[pl.ds(0, WRT)], acc_sh.at[pl.ds(zstart, WRT)])

    @pl.when(s == NTILES - 1)
    def _():
        pltpu.sync_copy(zeros_hbm.at[pl.ds(0, WTAIL)],
                        acc_sh.at[pl.ds(NTILES * WRT, WTAIL)])

    pltpu.sync_copy(src_hbm.at[r, s], sidx2.at[0])
    pltpu.sync_copy(dst_hbm.at[r, s], didx2.at[0])
    plsc.subcore_barrier()

    pltpu.async_copy(x_hbm.at[sidx2.at[0, 0]], rows2.at[0], gsem)
    pltpu.async_copy(x_hbm.at[sidx2.at[0, 1]], rows2.at[1], gsem)

    def _step(t, carry):
        i, j = lax.div(t, GRP), lax.rem(t, GRP)
        b = lax.rem(t, 3)
        pltpu.make_async_copy(x_hbm.at[pl.ds(0, CHUNK)],
                              rows2.at[b], gsem).wait()
        pltpu.async_copy(rows2.at[b], acc_sh.at[didx2.at[lax.rem(i, 2), j]],
                         ssem, add=True)

        @pl.when(t + 2 < t_total)
        def _():
            @pl.when(t >= 1)
            def _():
                pltpu.make_async_copy(x_hbm.at[pl.ds(0, CHUNK)],
                                      rows2.at[0], ssem).wait()

            @pl.when(jnp.logical_and(j == 0, i + 1 < n_my))
            def _():
                g = (i + 1) * NTILES + s
                ip = lax.rem(i + 1, 2)
                pltpu.async_copy(src_hbm.at[r, g], sidx2.at[ip], isem)
                pltpu.async_copy(dst_hbm.at[r, g], didx2.at[ip], isem)

            tn = t + 2
            i2, j2 = lax.div(tn, GRP), lax.rem(tn, GRP)

            @pl.when(j2 == 0)
            def _():
                pltpu.make_async_copy(src_hbm.at[r, s], sidx2.at[0],
                                      isem).wait()
                pltpu.make_async_copy(dst_hbm.at[r, s], didx2.at[0],
                                      isem).wait()

            pltpu.async_copy(x_hbm.at[sidx2.at[lax.rem(i2, 2), j2]],
                             rows2.at[lax.rem(tn, 3)], gsem)

        return carry

    lax.fori_loop(0, t_total, _step, 0)
    for _ in range(3):
        pltpu.make_async_copy(x_hbm.at[pl.ds(0, CHUNK)],
                              rows2.at[0], ssem).wait()
    plsc.subcore_barrier()

    pltpu.sync_copy(acc_sh.at[pl.ds(zstart, WRT)],
                    out_hbm.at[r, pl.ds(zstart, WRT)])

    @pl.when(s == NTILES - 1)
    def _():
        pltpu.sync_copy(acc_sh.at[pl.ds(NTILES * WRT, WTAIL)],
                        out_hbm.at[r, pl.ds(NTILES * WRT, WTAIL)])


_sc_agg = functools.partial(
    pl.kernel,
    out_type=jax.ShapeDtypeStruct((2, N, D), jnp.float32),
    mesh=plsc.VectorSubcoreMesh(core_axis_name="c", subcore_axis_name="s"),
    scratch_types=[
        pltpu.VMEM((2, GRP, CHUNK), jnp.int32),
        pltpu.VMEM((2, GRP, CHUNK), jnp.int32),
        pltpu.VMEM((3, CHUNK, D), jnp.float32),
        pltpu.VMEM_SHARED((N_ACC, D), jnp.float32),
        pltpu.SemaphoreType.DMA,
        pltpu.SemaphoreType.DMA,
        pltpu.SemaphoreType.DMA,
    ],
)(_sc_agg_body)


RB = 1000


def _tc_body(aggj_ref, xj_ref, aggr_ref, xr_ref,
             wrelj_ref, wrootj_ref, bj_ref,
             wrelr_ref, wrootr_ref, br_ref,
             wm1a_ref, wm1b_ref, bm1_ref,
             wm2_ref, bm2_ref, wm3_ref, bm3_ref, out_ref):
    f32 = jnp.float32
    hj = (jnp.dot(aggj_ref[...], wrelj_ref[...], preferred_element_type=f32)
          + jnp.dot(xj_ref[...], wrootj_ref[...], preferred_element_type=f32)
          + bj_ref[...])
    hj = jnp.maximum(hj, 0.0)
    hr = (jnp.dot(aggr_ref[...], wrelr_ref[...], preferred_element_type=f32)
          + jnp.dot(xr_ref[...], wrootr_ref[...], preferred_element_type=f32)
          + br_ref[...])
    hr = jnp.maximum(hr, 0.0)
    h1 = (jnp.dot(hj, wm1a_ref[...], preferred_element_type=f32)
          + jnp.dot(hr, wm1b_ref[...], preferred_element_type=f32)
          + bm1_ref[...])
    h1 = jnp.maximum(h1, 0.0)
    h2 = jnp.maximum(
        jnp.dot(h1, wm2_ref[...], preferred_element_type=f32) + bm2_ref[...],
        0.0)
    out_ref[...] = (jnp.sum(h2 * wm3_ref[...], axis=1, keepdims=True)
                    + bm3_ref[...])


def _full_spec(shape):
    return pl.BlockSpec(shape, lambda i: (0,) * len(shape))


_tc_epilogue = pl.pallas_call(
    _tc_body,
    grid=(N // RB,),
    in_specs=[
        pl.BlockSpec((RB, D), lambda i: (i, 0)),
        pl.BlockSpec((RB, D), lambda i: (i, 0)),
        pl.BlockSpec((RB, D), lambda i: (i, 0)),
        pl.BlockSpec((RB, D), lambda i: (i, 0)),
        _full_spec((D, D)), _full_spec((D, D)), _full_spec((1, D)),
        _full_spec((D, D)), _full_spec((D, D)), _full_spec((1, D)),
        _full_spec((D, H1)), _full_spec((D, H1)), _full_spec((1, H1)),
        _full_spec((H1, H2)), _full_spec((1, H2)),
        _full_spec((1, H2)), _full_spec((1, 1)),
    ],
    out_specs=pl.BlockSpec((RB, 1), lambda i: (i, 0)),
    out_shape=jax.ShapeDtypeStruct((N, 1), jnp.float32),
)


def _pad_idx(idx, fill):
    pad = jnp.full((E_PAD - E,), fill, jnp.int32)
    return jnp.concatenate([idx, pad]).reshape(NGRP, GRP, CHUNK)


def kernel(x_skill, x_job, x_resume, edge_index_skill_job,
           edge_index_skill_resume, W_rel_sj, b_rel_sj, W_root_sj, W_rel_sr,
           b_rel_sr, W_root_sr, Wm1, bm1, Wm2, bm2, Wm3, bm3):
    src_all = jnp.stack([_pad_idx(edge_index_skill_job[0], 0),
                         _pad_idx(edge_index_skill_resume[0], 0)])
    dst_all = jnp.stack([_pad_idx(edge_index_skill_job[1], N),
                         _pad_idx(edge_index_skill_resume[1], N)])
    zeros = jnp.zeros((WRT, D), jnp.float32)
    agg = _sc_agg(x_skill, src_all, dst_all, zeros)

    out = _tc_epilogue(
        agg[0], x_job, agg[1], x_resume,
        W_rel_sj.T, W_root_sj.T, b_rel_sj.reshape(1, D),
        W_rel_sr.T, W_root_sr.T, b_rel_sr.reshape(1, D),
        Wm1.T[:D], Wm1.T[D:], bm1.reshape(1, H1),
        Wm2.T, bm2.reshape(1, H2),
        Wm3, bm3.reshape(1, 1),
    )
    return out.reshape(N)


# speedup vs baseline: 13.2865x; 1.3778x over previous
import functools

import jax
import jax.numpy as jnp
from jax import lax
from jax.experimental import pallas as pl
from jax.experimental.pallas import tpu as pltpu
from jax.experimental.pallas import tpu_sc as plsc

N = 10000
E = 320000
D = 128
H1 = 512
H2 = 256

CHUNK = 128
GRP = 2
NGRP = E // (CHUNK * GRP)
NTILES = 16
WRT = 624
WTAIL = N - NTILES * WRT


def _sc_agg_body(x_hbm, edge_sj_hbm, edge_sr_hbm, zeros_hbm, out_hbm,
                 eidx, rows3, acc_sh, isem, gsem, ssem):
    r = lax.axis_index("c")
    s = lax.axis_index("s")
    n_my = (NGRP - 1 - s) // NTILES + 1
    t_total = n_my * GRP

    zstart = s * WRT
    pltpu.sync_copy(zeros_hbm.at[pl.ds(0, WRT)], acc_sh.at[pl.ds(zstart, WRT)])

    @pl.when(s == NTILES - 1)
    def _():
        pltpu.sync_copy(zeros_hbm.at[pl.ds(0, WTAIL)],
                        acc_sh.at[pl.ds(NTILES * WRT, WTAIL)])

    def _run(edge_hbm):
        pltpu.sync_copy(edge_hbm.at[:, s], eidx.at[0])
        pltpu.sync_copy(edge_hbm.at[:, NTILES + s], eidx.at[1])
        plsc.subcore_barrier()

        pltpu.async_copy(x_hbm.at[eidx.at[0, 0, 0]], rows3.at[0], gsem)
        pltpu.async_copy(x_hbm.at[eidx.at[0, 0, 1]], rows3.at[1], gsem)

        def _step(t, carry):
            i, j = lax.div(t, GRP), lax.rem(t, GRP)
            b = lax.rem(t, 3)
            ib = lax.rem(i, 3)
            pltpu.make_async_copy(x_hbm.at[pl.ds(0, CHUNK)],
                                  rows3.at[b], gsem).wait()
            pltpu.async_copy(rows3.at[b], acc_sh.at[eidx.at[ib, 1, j]],
                             ssem, add=True)

            @pl.when(t + 2 < t_total)
            def _():
                @pl.when(t >= 1)
                def _():
                    pltpu.make_async_copy(x_hbm.at[pl.ds(0, CHUNK)],
                                          rows3.at[0], ssem).wait()

                @pl.when(jnp.logical_and(j == 0, i + 2 < n_my))
                def _():
                    g = (i + 2) * NTILES + s
                    pltpu.async_copy(edge_hbm.at[:, g],
                                     eidx.at[lax.rem(i + 2, 3)], isem)

                tn = t + 2
                i2, j2 = lax.div(tn, GRP), lax.rem(tn, GRP)

                @pl.when(jnp.logical_and(j2 == 0, i2 >= 2))
                def _():
                    pltpu.make_async_copy(edge_hbm.at[:, s], eidx.at[0],
                                          isem).wait()

                pltpu.async_copy(x_hbm.at[eidx.at[lax.rem(i2, 3), 0, j2]],
                                 rows3.at[lax.rem(tn, 3)], gsem)

            return carry

        lax.fori_loop(0, t_total, _step, 0)
        for _ in range(3):
            pltpu.make_async_copy(x_hbm.at[pl.ds(0, CHUNK)],
                                  rows3.at[0], ssem).wait()

    @pl.when(r == 0)
    def _():
        _run(edge_sj_hbm)

    @pl.when(r == 1)
    def _():
        _run(edge_sr_hbm)

    plsc.subcore_barrier()

    pltpu.sync_copy(acc_sh.at[pl.ds(zstart, WRT)],
                    out_hbm.at[r, pl.ds(zstart, WRT)])

    @pl.when(s == NTILES - 1)
    def _():
        pltpu.sync_copy(acc_sh.at[pl.ds(NTILES * WRT, WTAIL)],
                        out_hbm.at[r, pl.ds(NTILES * WRT, WTAIL)])


_sc_agg = functools.partial(
    pl.kernel,
    out_type=jax.ShapeDtypeStruct((2, N, D), jnp.float32),
    mesh=plsc.VectorSubcoreMesh(core_axis_name="c", subcore_axis_name="s"),
    scratch_types=[
        pltpu.VMEM((3, 2, GRP, CHUNK), jnp.int32),
        pltpu.VMEM((3, CHUNK, D), jnp.float32),
        pltpu.VMEM_SHARED((N, D), jnp.float32),
        pltpu.SemaphoreType.DMA,
        pltpu.SemaphoreType.DMA,
        pltpu.SemaphoreType.DMA,
    ],
)(_sc_agg_body)


RB = 1000


def _tc_body(aggj_ref, xj_ref, aggr_ref, xr_ref,
             wrelj_ref, wrootj_ref, bj_ref,
             wrelr_ref, wrootr_ref, br_ref,
             wm1a_ref, wm1b_ref, bm1_ref,
             wm2_ref, bm2_ref, wm3_ref, bm3_ref, out_ref):
    f32 = jnp.float32
    hj = (jnp.dot(aggj_ref[...], wrelj_ref[...], preferred_element_type=f32)
          + jnp.dot(xj_ref[...], wrootj_ref[...], preferred_element_type=f32)
          + bj_ref[...])
    hj = jnp.maximum(hj, 0.0)
    hr = (jnp.dot(aggr_ref[...], wrelr_ref[...], preferred_element_type=f32)
          + jnp.dot(xr_ref[...], wrootr_ref[...], preferred_element_type=f32)
          + br_ref[...])
    hr = jnp.maximum(hr, 0.0)
    h1 = (jnp.dot(hj, wm1a_ref[...], preferred_element_type=f32)
          + jnp.dot(hr, wm1b_ref[...], preferred_element_type=f32)
          + bm1_ref[...])
    h1 = jnp.maximum(h1, 0.0)
    h2 = jnp.maximum(
        jnp.dot(h1, wm2_ref[...], preferred_element_type=f32) + bm2_ref[...],
        0.0)
    out_ref[...] = (jnp.sum(h2 * wm3_ref[...], axis=1, keepdims=True)
                    + bm3_ref[...])


def _full_spec(shape):
    return pl.BlockSpec(shape, lambda i: (0,) * len(shape))


_tc_epilogue = pl.pallas_call(
    _tc_body,
    grid=(N // RB,),
    in_specs=[
        pl.BlockSpec((RB, D), lambda i: (i, 0)),
        pl.BlockSpec((RB, D), lambda i: (i, 0)),
        pl.BlockSpec((RB, D), lambda i: (i, 0)),
        pl.BlockSpec((RB, D), lambda i: (i, 0)),
        _full_spec((D, D)), _full_spec((D, D)), _full_spec((1, D)),
        _full_spec((D, D)), _full_spec((D, D)), _full_spec((1, D)),
        _full_spec((D, H1)), _full_spec((D, H1)), _full_spec((1, H1)),
        _full_spec((H1, H2)), _full_spec((1, H2)),
        _full_spec((1, H2)), _full_spec((1, 1)),
    ],
    out_specs=pl.BlockSpec((RB, 1), lambda i: (i, 0)),
    out_shape=jax.ShapeDtypeStruct((N, 1), jnp.float32),
)


def kernel(x_skill, x_job, x_resume, edge_index_skill_job,
           edge_index_skill_resume, W_rel_sj, b_rel_sj, W_root_sj, W_rel_sr,
           b_rel_sr, W_root_sr, Wm1, bm1, Wm2, bm2, Wm3, bm3):
    edge_sj = edge_index_skill_job.reshape(2, NGRP, GRP, CHUNK)
    edge_sr = edge_index_skill_resume.reshape(2, NGRP, GRP, CHUNK)
    zeros = jnp.zeros((WRT, D), jnp.float32)
    agg = _sc_agg(x_skill, edge_sj, edge_sr, zeros)

    out = _tc_epilogue(
        agg[0], x_job, agg[1], x_resume,
        W_rel_sj.T, W_root_sj.T, b_rel_sj.reshape(1, D),
        W_rel_sr.T, W_root_sr.T, b_rel_sr.reshape(1, D),
        Wm1.T[:D], Wm1.T[D:], bm1.reshape(1, H1),
        Wm2.T, bm2.reshape(1, H2),
        Wm3, bm3.reshape(1, 1),
    )
    return out.reshape(N)
